# Initial kernel scaffold; baseline (speedup 1.0000x reference)
import functools

import jax
import jax.numpy as jnp
from jax import lax
from jax.experimental import pallas as pl
from jax.experimental.pallas import tpu as pltpu
from jax.experimental.pallas import tpu_sc as plsc

N = 10000
E = 320000
D = 128
NC = 2
NS = 16
NW = NC * NS
EW = E // NW
CHUNK = 128
NFULL = EW // CHUNK
REM = EW - NFULL * CHUNK
RPS = N // NS
S2_CHUNK = 624
S2_LAST = N - 15 * S2_CHUNK

_mesh = plsc.VectorSubcoreMesh(
    core_axis_name="c", subcore_axis_name="s", num_cores=NC, num_subcores=NS
)


def _sc_segsum_wide(x_hbm, src_hbm, dst_hbm, out_hbm, acc, sidx, didx,
                    sidxr, didxr, rows, sem):
    c = lax.axis_index("c")
    s = lax.axis_index("s")
    wid = s * NC + c
    base = wid * EW

    zero16 = jnp.zeros((16,), jnp.float32)

    def _zrow(i, carry):
        for k in range(D // 16):
            rows[i, pl.ds(k * 16, 16)] = zero16
        return carry

    lax.fori_loop(0, CHUNK, _zrow, 0)
    r0 = s * RPS
    off = 0
    for m in (128, 128, 128, 128, RPS - 4 * 128):
        pltpu.sync_copy(rows.at[pl.ds(0, m)], acc.at[pl.ds(r0 + off, m)])
        off += m
    plsc.subcore_barrier()

    def _step_full(j, carry):
        e0 = base + j * CHUNK
        pltpu.sync_copy(src_hbm.at[pl.ds(e0, CHUNK)], sidx)
        pltpu.async_copy(x_hbm.at[sidx], rows, sem).wait()
        pltpu.sync_copy(dst_hbm.at[pl.ds(e0, CHUNK)], didx)
        pltpu.sync_copy(rows, acc.at[didx], add=True)
        return carry

    lax.fori_loop(0, NFULL, _step_full, 0)

    if REM:
        e0 = base + NFULL * CHUNK
        pltpu.sync_copy(src_hbm.at[pl.ds(e0, REM)], sidxr)
        pltpu.async_copy(x_hbm.at[sidxr], rows.at[pl.ds(0, REM)], sem).wait()
        pltpu.sync_copy(dst_hbm.at[pl.ds(e0, REM)], didxr)
        pltpu.sync_copy(rows.at[pl.ds(0, REM)], acc.at[didxr], add=True)

    plsc.subcore_barrier()
    pltpu.sync_copy(acc.at[pl.ds(r0, RPS)], out_hbm.at[c, pl.ds(r0, RPS)])


_sc1 = pl.kernel(
    _sc_segsum_wide,
    out_type=jax.ShapeDtypeStruct((NC, N, D), jnp.float32),
    mesh=_mesh,
    scratch_types=[
        pltpu.VMEM_SHARED((N, D), jnp.float32),
        pltpu.VMEM((CHUNK,), jnp.int32),
        pltpu.VMEM((CHUNK,), jnp.int32),
        pltpu.VMEM((REM,), jnp.int32),
        pltpu.VMEM((REM,), jnp.int32),
        pltpu.VMEM((CHUNK, D), jnp.float32),
        pltpu.SemaphoreType.DMA,
    ],
)


def _sc_segsum_scalar(y_hbm, src_hbm, dst_hbm, out_hbm, acc, sidx, didx,
                      sidxr, didxr, yv, zbuf, sem):
    c = lax.axis_index("c")
    s = lax.axis_index("s")
    wid = s * NC + c
    base = wid * EW

    zero16 = jnp.zeros((16,), jnp.float32)

    def _z(i, carry):
        zbuf[pl.ds(i * 16, 16)] = zero16
        return carry

    lax.fori_loop(0, S2_LAST // 16, _z, 0)

    @pl.when(s < NS - 1)
    def _():
        pltpu.sync_copy(zbuf.at[pl.ds(0, S2_CHUNK)],
                        acc.at[pl.ds(s * S2_CHUNK, S2_CHUNK)])

    @pl.when(s == NS - 1)
    def _():
        pltpu.sync_copy(zbuf, acc.at[pl.ds((NS - 1) * S2_CHUNK, S2_LAST)])

    plsc.subcore_barrier()

    def _step_full(j, carry):
        e0 = base + j * CHUNK
        pltpu.sync_copy(src_hbm.at[pl.ds(e0, CHUNK)], sidx)
        pltpu.async_copy(y_hbm.at[sidx], yv, sem).wait()
        pltpu.sync_copy(dst_hbm.at[pl.ds(e0, CHUNK)], didx)
        pltpu.sync_copy(yv, acc.at[didx], add=True)
        return carry

    lax.fori_loop(0, NFULL, _step_full, 0)

    if REM:
        e0 = base + NFULL * CHUNK
        pltpu.sync_copy(src_hbm.at[pl.ds(e0, REM)], sidxr)
        pltpu.async_copy(y_hbm.at[sidxr], yv.at[pl.ds(0, REM)], sem).wait()
        pltpu.sync_copy(dst_hbm.at[pl.ds(e0, REM)], didxr)
        pltpu.sync_copy(yv.at[pl.ds(0, REM)], acc.at[didxr], add=True)

    plsc.subcore_barrier()

    @pl.when(s < NS - 1)
    def _():
        pltpu.sync_copy(acc.at[pl.ds(s * S2_CHUNK, S2_CHUNK)],
                        out_hbm.at[c, pl.ds(s * S2_CHUNK, S2_CHUNK)])

    @pl.when(s == NS - 1)
    def _():
        pltpu.sync_copy(acc.at[pl.ds((NS - 1) * S2_CHUNK, S2_LAST)],
                        out_hbm.at[c, pl.ds((NS - 1) * S2_CHUNK, S2_LAST)])


_sc2 = pl.kernel(
    _sc_segsum_scalar,
    out_type=jax.ShapeDtypeStruct((NC, N), jnp.float32),
    mesh=_mesh,
    scratch_types=[
        pltpu.VMEM_SHARED((N,), jnp.float32),
        pltpu.VMEM((CHUNK,), jnp.int32),
        pltpu.VMEM((CHUNK,), jnp.int32),
        pltpu.VMEM((REM,), jnp.int32),
        pltpu.VMEM((REM,), jnp.int32),
        pltpu.VMEM((CHUNK,), jnp.float32),
        pltpu.VMEM((S2_LAST,), jnp.float32),
        pltpu.SemaphoreType.DMA,
    ],
)

_BM = 1000


def _tc_dense_body(p0, p1, x, w1rel, w1root, b1, w2rel_t, w2root_t,
                   y_out, r2_out):
    agg = p0[...] + p1[...]
    h = jnp.dot(agg, w1rel[...], preferred_element_type=jnp.float32)
    h = h + jnp.dot(x[...], w1root[...], preferred_element_type=jnp.float32)
    h = jnp.maximum(h + b1[...], 0.0)
    y_out[...] = jnp.sum(h * w2rel_t[...], axis=1, keepdims=True)
    r2_out[...] = jnp.sum(h * w2root_t[...], axis=1, keepdims=True)


def _tc_out_body(s0, s1, r2, b2, o):
    o[...] = jax.nn.sigmoid(s0[...] + s1[...] + r2[...] + b2[...])


def kernel(x, edge_index, W1_rel, W1_root, b1, W2_rel, W2_root, b2):
    src = edge_index[0]
    dst = edge_index[1]

    parts = _sc1(x, src, dst)

    full = pl.BlockSpec((D, D), lambda i: (0, 0))
    row1 = pl.BlockSpec((1, D), lambda i: (0, 0))
    blk = pl.BlockSpec((_BM, D), lambda i: (i, 0))
    col = pl.BlockSpec((_BM, 1), lambda i: (i, 0))
    y, r2 = pl.pallas_call(
        _tc_dense_body,
        grid=(N // _BM,),
        in_specs=[blk, blk, blk, full, full, row1, row1, row1],
        out_specs=[col, col],
        out_shape=[
            jax.ShapeDtypeStruct((N, 1), jnp.float32),
            jax.ShapeDtypeStruct((N, 1), jnp.float32),
        ],
    )(parts[0], parts[1], x, W1_rel, W1_root, b1.reshape(1, D),
      W2_rel.reshape(1, D), W2_root.reshape(1, D))

    sparts = _sc2(y.reshape(N), src, dst)

    one = pl.BlockSpec((1, 1), lambda i: (0, 0))
    out = pl.pallas_call(
        _tc_out_body,
        grid=(N // _BM,),
        in_specs=[col, col, col, one],
        out_specs=col,
        out_shape=jax.ShapeDtypeStruct((N, 1), jnp.float32),
    )(sparts[0].reshape(N, 1), sparts[1].reshape(N, 1), r2,
      b2.reshape(1, 1))
    return out


# speedup vs baseline: 7.3581x; 7.3581x over previous
import functools

import jax
import jax.numpy as jnp
from jax import lax
from jax.experimental import pallas as pl
from jax.experimental.pallas import tpu as pltpu
from jax.experimental.pallas import tpu_sc as plsc

N = 10000
E = 320000
D = 128
NC = 2
NS = 16
NW = NC * NS
EW = E // NW
CHUNK = 128
NFULL = EW // CHUNK
REM = EW - NFULL * CHUNK
RPS = 624
RPS_LAST = N - 15 * RPS
S2_CHUNK = RPS
S2_LAST = RPS_LAST

_mesh = plsc.VectorSubcoreMesh(
    core_axis_name="c", subcore_axis_name="s", num_cores=NC, num_subcores=NS
)


def _sc_segsum_wide(x_hbm, src_hbm, dst_hbm, out_hbm, acc, sidx, didx,
                    sidxr, didxr, rows, sem):
    c = lax.axis_index("c")
    s = lax.axis_index("s")
    wid = s * NC + c
    base = wid * EW

    zero16 = jnp.zeros((16,), jnp.float32)

    def _zrow(i, carry):
        for k in range(D // 16):
            rows[i, pl.ds(k * 16, 16)] = zero16
        return carry

    lax.fori_loop(0, CHUNK, _zrow, 0)
    r0 = s * RPS

    @pl.when(s < NS - 1)
    def _():
        off = 0
        for m in (128, 128, 128, 128, RPS - 4 * 128):
            pltpu.sync_copy(rows.at[pl.ds(0, m)], acc.at[pl.ds(r0 + off, m)])
            off += m

    @pl.when(s == NS - 1)
    def _():
        for k in range(RPS_LAST // CHUNK):
            pltpu.sync_copy(rows, acc.at[pl.ds(15 * RPS + k * CHUNK, CHUNK)])

    plsc.subcore_barrier()

    def _step_full(j, carry):
        e0 = base + j * CHUNK
        pltpu.sync_copy(src_hbm.at[pl.ds(e0, CHUNK)], sidx)
        pltpu.async_copy(x_hbm.at[sidx], rows, sem).wait()
        pltpu.sync_copy(dst_hbm.at[pl.ds(e0, CHUNK)], didx)
        pltpu.sync_copy(rows, acc.at[didx], add=True)
        return carry

    lax.fori_loop(0, NFULL, _step_full, 0)

    if REM:
        e0 = base + NFULL * CHUNK
        pltpu.sync_copy(src_hbm.at[pl.ds(e0, REM)], sidxr)
        pltpu.async_copy(x_hbm.at[sidxr], rows.at[pl.ds(0, REM)], sem).wait()
        pltpu.sync_copy(dst_hbm.at[pl.ds(e0, REM)], didxr)
        pltpu.sync_copy(rows.at[pl.ds(0, REM)], acc.at[didxr], add=True)

    plsc.subcore_barrier()

    @pl.when(s < NS - 1)
    def _():
        pltpu.sync_copy(acc.at[pl.ds(r0, RPS)], out_hbm.at[c, pl.ds(r0, RPS)])

    @pl.when(s == NS - 1)
    def _():
        pltpu.sync_copy(acc.at[pl.ds(15 * RPS, RPS_LAST)],
                        out_hbm.at[c, pl.ds(15 * RPS, RPS_LAST)])


_sc1 = pl.kernel(
    _sc_segsum_wide,
    out_type=jax.ShapeDtypeStruct((NC, N, D), jnp.float32),
    mesh=_mesh,
    scratch_types=[
        pltpu.VMEM_SHARED((N, D), jnp.float32),
        pltpu.VMEM((CHUNK,), jnp.int32),
        pltpu.VMEM((CHUNK,), jnp.int32),
        pltpu.VMEM((REM,), jnp.int32),
        pltpu.VMEM((REM,), jnp.int32),
        pltpu.VMEM((CHUNK, D), jnp.float32),
        pltpu.SemaphoreType.DMA,
    ],
)


def _sc_segsum_scalar(y_hbm, src_hbm, dst_hbm, out_hbm, acc, sidx, didx,
                      sidxr, didxr, yv, zbuf, sem):
    c = lax.axis_index("c")
    s = lax.axis_index("s")
    wid = s * NC + c
    base = wid * EW

    zero16 = jnp.zeros((16,), jnp.float32)

    def _z(i, carry):
        zbuf[pl.ds(i * 16, 16)] = zero16
        return carry

    lax.fori_loop(0, S2_LAST // 16, _z, 0)

    @pl.when(s < NS - 1)
    def _():
        pltpu.sync_copy(zbuf.at[pl.ds(0, S2_CHUNK)],
                        acc.at[pl.ds(s * S2_CHUNK, S2_CHUNK)])

    @pl.when(s == NS - 1)
    def _():
        pltpu.sync_copy(zbuf, acc.at[pl.ds((NS - 1) * S2_CHUNK, S2_LAST)])

    plsc.subcore_barrier()

    def _step_full(j, carry):
        e0 = base + j * CHUNK
        pltpu.sync_copy(src_hbm.at[pl.ds(e0, CHUNK)], sidx)
        pltpu.async_copy(y_hbm.at[sidx], yv, sem).wait()
        pltpu.sync_copy(dst_hbm.at[pl.ds(e0, CHUNK)], didx)
        pltpu.sync_copy(yv, acc.at[didx], add=True)
        return carry

    lax.fori_loop(0, NFULL, _step_full, 0)

    if REM:
        e0 = base + NFULL * CHUNK
        pltpu.sync_copy(src_hbm.at[pl.ds(e0, REM)], sidxr)
        pltpu.async_copy(y_hbm.at[sidxr], yv.at[pl.ds(0, REM)], sem).wait()
        pltpu.sync_copy(dst_hbm.at[pl.ds(e0, REM)], didxr)
        pltpu.sync_copy(yv.at[pl.ds(0, REM)], acc.at[didxr], add=True)

    plsc.subcore_barrier()

    @pl.when(s < NS - 1)
    def _():
        pltpu.sync_copy(acc.at[pl.ds(s * S2_CHUNK, S2_CHUNK)],
                        zbuf.at[pl.ds(0, S2_CHUNK)])
        pltpu.sync_copy(zbuf.at[pl.ds(0, S2_CHUNK)],
                        out_hbm.at[pl.ds(c * N + s * S2_CHUNK, S2_CHUNK)])

    @pl.when(s == NS - 1)
    def _():
        pltpu.sync_copy(acc.at[pl.ds((NS - 1) * S2_CHUNK, S2_LAST)], zbuf)
        pltpu.sync_copy(zbuf,
                        out_hbm.at[pl.ds(c * N + (NS - 1) * S2_CHUNK, S2_LAST)])


_sc2 = pl.kernel(
    _sc_segsum_scalar,
    out_type=jax.ShapeDtypeStruct((NC * N,), jnp.float32),
    mesh=_mesh,
    scratch_types=[
        pltpu.VMEM_SHARED((N,), jnp.float32),
        pltpu.VMEM((CHUNK,), jnp.int32),
        pltpu.VMEM((CHUNK,), jnp.int32),
        pltpu.VMEM((REM,), jnp.int32),
        pltpu.VMEM((REM,), jnp.int32),
        pltpu.VMEM((CHUNK,), jnp.float32),
        pltpu.VMEM((S2_LAST,), jnp.float32),
        pltpu.SemaphoreType.DMA,
    ],
)

_BM = 1000


def _tc_dense_body(p0, p1, x, w1rel, w1root, b1, w2rel_t, w2root_t,
                   y_out, r2_out):
    agg = p0[...] + p1[...]
    h = jnp.dot(agg, w1rel[...], preferred_element_type=jnp.float32)
    h = h + jnp.dot(x[...], w1root[...], preferred_element_type=jnp.float32)
    h = jnp.maximum(h + b1[...], 0.0)
    y_out[...] = jnp.sum(h * w2rel_t[...], axis=1, keepdims=True)
    r2_out[...] = jnp.sum(h * w2root_t[...], axis=1, keepdims=True)


def _tc_out_body(s0, s1, r2, b2, o):
    o[...] = jax.nn.sigmoid(s0[...] + s1[...] + r2[...] + b2[...])


def kernel(x, edge_index, W1_rel, W1_root, b1, W2_rel, W2_root, b2):
    src = edge_index[0]
    dst = edge_index[1]

    parts = _sc1(x, src, dst)

    full = pl.BlockSpec((D, D), lambda i: (0, 0))
    row1 = pl.BlockSpec((1, D), lambda i: (0, 0))
    blk = pl.BlockSpec((_BM, D), lambda i: (i, 0))
    col = pl.BlockSpec((_BM, 1), lambda i: (i, 0))
    y, r2 = pl.pallas_call(
        _tc_dense_body,
        grid=(N // _BM,),
        in_specs=[blk, blk, blk, full, full, row1, row1, row1],
        out_specs=[col, col],
        out_shape=[
            jax.ShapeDtypeStruct((N, 1), jnp.float32),
            jax.ShapeDtypeStruct((N, 1), jnp.float32),
        ],
    )(parts[0], parts[1], x, W1_rel, W1_root, b1.reshape(1, D),
      W2_rel.reshape(1, D), W2_root.reshape(1, D))

    sparts = _sc2(y.reshape(N), src, dst)

    one = pl.BlockSpec((1, 1), lambda i: (0, 0))
    out = pl.pallas_call(
        _tc_out_body,
        grid=(N // _BM,),
        in_specs=[col, col, col, one],
        out_specs=col,
        out_shape=jax.ShapeDtypeStruct((N, 1), jnp.float32),
    )(sparts[:N].reshape(N, 1), sparts[N:].reshape(N, 1), r2,
      b2.reshape(1, 1))
    return out
